# baseline (speedup 1.0000x reference)
import functools

import jax
import jax.numpy as jnp
from jax import lax
from jax.experimental import pallas as pl
from jax.experimental.pallas import tpu as pltpu

_KNOCK = 1e30
_P = 1024
_K = 20
_NC = 10
_Q = 256
_VLIM = 48 * 1024 * 1024


def _dist_matrix(x_full, x_q):
    f32 = jnp.float32
    xsq = x_full * x_full
    sq_row = lax.dot_general(jnp.ones((1, x_full.shape[1]), f32), xsq,
                             (((1,), (1,)), ((), ())),
                             preferred_element_type=f32)
    gram = lax.dot_general(x_q, x_full, (((1,), (1,)), ((), ())),
                           preferred_element_type=f32)
    return sq_row - 2.0 * gram


_SUB = 16


def _knn_onehot(x_full, x_q, *, k, out_dtype):
    f32 = jnp.float32
    P = x_full.shape[0]
    Q = x_q.shape[0]
    dist = _dist_matrix(x_full, x_q)
    lane = lax.broadcasted_iota(jnp.int32, (_SUB, P), 1).astype(f32)
    nsb = Q // _SUB
    picks = [[None] * nsb for _ in range(k)]
    for sb in range(nsb):
        dwork = dist[sb * _SUB:(sb + 1) * _SUB]
        for r in range(k):
            rmin = jnp.min(dwork, axis=1, keepdims=True)
            first = jnp.min(jnp.where(dwork == rmin, lane, float(P)),
                            axis=1, keepdims=True)
            pick = lane == first
            dwork = jnp.where(pick, _KNOCK, dwork)
            picks[r][sb] = pick.astype(out_dtype)
    return jnp.concatenate([p for row in picks for p in row], axis=0)


def _edge_conv(x_full, x_q, wc_ref, wn_ref, b1_ref, w2_ref, b2_ref,
               w3_ref, b3_ref, *, k, mxu_bf16):
    f32 = jnp.float32
    mxu = jnp.bfloat16 if mxu_bf16 else f32
    q = x_q.shape[0]
    sel = _knn_onehot(x_full, x_q, k=k, out_dtype=mxu)
    xm = x_full.astype(mxu)
    n_full = jnp.dot(xm, wn_ref[...], preferred_element_type=f32).astype(mxu)
    ac = jnp.dot(x_q.astype(mxu), wc_ref[...],
                 preferred_element_type=f32) + b1_ref[...]
    nj = jnp.dot(sel, n_full, preferred_element_type=f32)
    h = (nj.reshape(k, q, -1) + ac[None, :, :]).reshape(k * q, -1)
    h = jnp.maximum(h, 0.0)
    h = jnp.maximum(jnp.dot(h.astype(mxu), w2_ref[...],
                            preferred_element_type=f32) + b2_ref[...], 0.0)
    h = jnp.maximum(jnp.dot(h.astype(mxu), w3_ref[...],
                            preferred_element_type=f32) + b3_ref[...], 0.0)
    return jnp.max(h.reshape(k, q, -1), axis=0)


def _conv1_kernel(x_ref, wc_ref, wn_ref, b1_ref, w2_ref, b2_ref, w3_ref,
                  b3_ref, o_ref, *, k, q):
    qc = pl.program_id(1)
    start = pl.multiple_of(qc * q, q)
    x_full = x_ref[0]
    x_q = x_ref[0, pl.ds(start, q), :]
    o_ref[0] = _edge_conv(x_full, x_q, wc_ref, wn_ref, b1_ref, w2_ref,
                          b2_ref, w3_ref, b3_ref, k=k, mxu_bf16=True)


def _conv2_head_kernel(x_ref, wc_ref, wn_ref, b1_ref, w2_ref, b2_ref, w3_ref,
                       b3_ref, w0_ref, b0_ref, l1w_ref, l1b_ref, l2w_ref,
                       l2b_ref, l3w_ref, l3b_ref, o_ref, pool_acc, *, k, q):
    f32 = jnp.float32
    bf = jnp.bfloat16
    qc = pl.program_id(1)
    start = pl.multiple_of(qc * q, q)
    x_full = x_ref[0]
    x_q = x_ref[0, pl.ds(start, q), :]
    feat = _edge_conv(x_full, x_q, wc_ref, wn_ref, b1_ref, w2_ref,
                      b2_ref, w3_ref, b3_ref, k=k, mxu_bf16=True)

    h0 = jnp.maximum(jnp.dot(feat.astype(bf), w0_ref[...],
                             preferred_element_type=f32) + b0_ref[...], 0.0)
    pooled = jnp.max(h0, axis=0, keepdims=True)

    @pl.when(qc == 0)
    def _():
        pool_acc[...] = pooled

    @pl.when(qc != 0)
    def _():
        pool_acc[...] = jnp.maximum(pool_acc[...], pooled)

    @pl.when(qc == pl.num_programs(1) - 1)
    def _():
        p = pool_acc[...]
        t1 = jnp.maximum(jnp.dot(p, l1w_ref[...], preferred_element_type=f32)
                         + l1b_ref[...], 0.0)
        t2 = jnp.maximum(jnp.dot(t1, l2w_ref[...], preferred_element_type=f32)
                         + l2b_ref[...], 0.0)
        logits = jnp.dot(t2, l3w_ref[...], preferred_element_type=f32) \
            + l3b_ref[...]
        m = jnp.max(logits, axis=1, keepdims=True)
        lse = m + jnp.log(jnp.sum(jnp.exp(logits - m), axis=1, keepdims=True))
        o_ref[0] = logits - lse


def _split_edge_weights(w1, c_in, dtype):
    w1a, w1b = w1[:c_in], w1[c_in:]
    return (w1a - w1b).astype(dtype), w1b.astype(dtype)


def kernel(pos, batch, c1w1, c1b1, c1w2, c1b2, c1w3, c1b3,
           c2w1, c2b1, c2w2, c2b2, c2w3, c2b3,
           w0, b0, l1w, l1b, l2w, l2b, l3w, l3b):
    del batch
    f32 = jnp.float32
    bf = jnp.bfloat16
    n, c = pos.shape
    B = n // _P
    x = pos.reshape(B, _P, c)

    wc1, wn1 = _split_edge_weights(c1w1, c, bf)
    conv1_w = (wc1, wn1, c1b1, c1w2.astype(bf), c1b2, c1w3.astype(bf), c1b3)
    h1 = c1w3.shape[1]

    grid1 = pltpu.PrefetchScalarGridSpec(
        num_scalar_prefetch=0,
        grid=(B, _P // _Q),
        in_specs=[pl.BlockSpec((1, _P, c), lambda b, qi: (b, 0, 0))]
                 + [pl.BlockSpec(w.shape, lambda b, qi: (0, 0)) for w in conv1_w],
        out_specs=pl.BlockSpec((1, _Q, h1), lambda b, qi: (b, qi, 0)),
    )
    x1 = pl.pallas_call(
        functools.partial(_conv1_kernel, k=_K, q=_Q),
        out_shape=jax.ShapeDtypeStruct((B, _P, h1), f32),
        grid_spec=grid1,
        compiler_params=pltpu.CompilerParams(
            dimension_semantics=("parallel", "parallel"),
            vmem_limit_bytes=_VLIM),
    )(x, *conv1_w)

    wc2, wn2 = _split_edge_weights(c2w1, c2w1.shape[0] // 2, bf)
    wc2, wn2 = wc2[:h1], wn2[:h1]
    conv2_w = (wc2, wn2, c2b1, c2w2.astype(bf), c2b2, c2w3.astype(bf), c2b3)
    head_w = (w0.astype(bf), b0, l1w, l1b, l2w, l2b, l3w, l3b)
    weights = conv2_w + head_w

    grid2 = pltpu.PrefetchScalarGridSpec(
        num_scalar_prefetch=0,
        grid=(B, _P // _Q),
        in_specs=[pl.BlockSpec((1, _P, h1), lambda b, qi: (b, 0, 0))]
                 + [pl.BlockSpec(w.shape, lambda b, qi: (0, 0)) for w in weights],
        out_specs=pl.BlockSpec((1, 1, _NC), lambda b, qi: (b, 0, 0)),
        scratch_shapes=[pltpu.VMEM((1, w0.shape[1]), f32)],
    )
    out = pl.pallas_call(
        functools.partial(_conv2_head_kernel, k=_K, q=_Q),
        out_shape=jax.ShapeDtypeStruct((B, 1, _NC), f32),
        grid_spec=grid2,
        compiler_params=pltpu.CompilerParams(
            dimension_semantics=("parallel", "arbitrary"),
            vmem_limit_bytes=_VLIM),
    )(x1, *weights)
    return out[:, 0, :]


# speedup vs baseline: 1.0232x; 1.0232x over previous
import functools

import jax
import jax.numpy as jnp
from jax import lax
from jax.experimental import pallas as pl
from jax.experimental.pallas import tpu as pltpu

_KNOCK = 1e30
_P = 1024
_K = 20
_NC = 10
_Q = 256
_Q2 = 512
_VLIM = 48 * 1024 * 1024


def _dist_matrix(x_full, x_q):
    f32 = jnp.float32
    xsq = x_full * x_full
    sq_row = lax.dot_general(jnp.ones((1, x_full.shape[1]), f32), xsq,
                             (((1,), (1,)), ((), ())),
                             preferred_element_type=f32)
    gram = lax.dot_general(x_q, x_full, (((1,), (1,)), ((), ())),
                           preferred_element_type=f32)
    return sq_row - 2.0 * gram


_SUB = 16


def _knn_onehot(x_full, x_q, *, k, out_dtype):
    f32 = jnp.float32
    P = x_full.shape[0]
    Q = x_q.shape[0]
    dist = _dist_matrix(x_full, x_q)
    lane = lax.broadcasted_iota(jnp.int32, (_SUB, P), 1).astype(f32)
    nsb = Q // _SUB
    picks = [[None] * nsb for _ in range(k)]
    for sb in range(nsb):
        dwork = dist[sb * _SUB:(sb + 1) * _SUB]
        for r in range(k):
            rmin = jnp.min(dwork, axis=1, keepdims=True)
            first = jnp.min(jnp.where(dwork == rmin, lane, float(P)),
                            axis=1, keepdims=True)
            pick = lane == first
            dwork = jnp.where(pick, _KNOCK, dwork)
            picks[r][sb] = pick.astype(out_dtype)
    return jnp.concatenate([p for row in picks for p in row], axis=0)


def _edge_conv(x_full, x_q, wc_ref, wn_ref, b1_ref, w2_ref, b2_ref,
               w3_ref, b3_ref, *, k, mxu_bf16):
    f32 = jnp.float32
    mxu = jnp.bfloat16 if mxu_bf16 else f32
    q = x_q.shape[0]
    sel = _knn_onehot(x_full, x_q, k=k, out_dtype=mxu)
    xm = x_full.astype(mxu)
    n_full = jnp.dot(xm, wn_ref[...], preferred_element_type=f32).astype(mxu)
    ac = jnp.dot(x_q.astype(mxu), wc_ref[...],
                 preferred_element_type=f32) + b1_ref[...]
    nj = jnp.dot(sel, n_full, preferred_element_type=f32)
    h = (nj.reshape(k, q, -1) + ac[None, :, :]).reshape(k * q, -1)
    h = jnp.maximum(h, 0.0)
    h = jnp.maximum(jnp.dot(h.astype(mxu), w2_ref[...],
                            preferred_element_type=f32) + b2_ref[...], 0.0)
    h = jnp.maximum(jnp.dot(h.astype(mxu), w3_ref[...],
                            preferred_element_type=f32) + b3_ref[...], 0.0)
    return jnp.max(h.reshape(k, q, -1), axis=0)


def _conv1_kernel(x_ref, wc_ref, wn_ref, b1_ref, w2_ref, b2_ref, w3_ref,
                  b3_ref, o_ref, *, k, q):
    qc = pl.program_id(1)
    start = pl.multiple_of(qc * q, q)
    x_full = x_ref[0]
    x_q = x_ref[0, pl.ds(start, q), :]
    o_ref[0] = _edge_conv(x_full, x_q, wc_ref, wn_ref, b1_ref, w2_ref,
                          b2_ref, w3_ref, b3_ref, k=k, mxu_bf16=False)


def _conv2_head_kernel(x_ref, wc_ref, wn_ref, b1_ref, w2_ref, b2_ref, w3_ref,
                       b3_ref, w0_ref, b0_ref, l1w_ref, l1b_ref, l2w_ref,
                       l2b_ref, l3w_ref, l3b_ref, o_ref, pool_acc, *, k, q):
    f32 = jnp.float32
    bf = jnp.bfloat16
    qc = pl.program_id(1)
    start = pl.multiple_of(qc * q, q)
    x_full = x_ref[0]
    x_q = x_ref[0, pl.ds(start, q), :]
    feat = _edge_conv(x_full, x_q, wc_ref, wn_ref, b1_ref, w2_ref,
                      b2_ref, w3_ref, b3_ref, k=k, mxu_bf16=True)

    h0 = jnp.maximum(jnp.dot(feat.astype(bf), w0_ref[...],
                             preferred_element_type=f32) + b0_ref[...], 0.0)
    pooled = jnp.max(h0, axis=0, keepdims=True)

    @pl.when(qc == 0)
    def _():
        pool_acc[...] = pooled

    @pl.when(qc != 0)
    def _():
        pool_acc[...] = jnp.maximum(pool_acc[...], pooled)

    @pl.when(qc == pl.num_programs(1) - 1)
    def _():
        p = pool_acc[...]
        t1 = jnp.maximum(jnp.dot(p, l1w_ref[...], preferred_element_type=f32)
                         + l1b_ref[...], 0.0)
        t2 = jnp.maximum(jnp.dot(t1, l2w_ref[...], preferred_element_type=f32)
                         + l2b_ref[...], 0.0)
        logits = jnp.dot(t2, l3w_ref[...], preferred_element_type=f32) \
            + l3b_ref[...]
        m = jnp.max(logits, axis=1, keepdims=True)
        lse = m + jnp.log(jnp.sum(jnp.exp(logits - m), axis=1, keepdims=True))
        o_ref[0] = logits - lse


def _split_edge_weights(w1, c_in, dtype):
    w1a, w1b = w1[:c_in], w1[c_in:]
    return (w1a - w1b).astype(dtype), w1b.astype(dtype)


def kernel(pos, batch, c1w1, c1b1, c1w2, c1b2, c1w3, c1b3,
           c2w1, c2b1, c2w2, c2b2, c2w3, c2b3,
           w0, b0, l1w, l1b, l2w, l2b, l3w, l3b):
    del batch
    f32 = jnp.float32
    bf = jnp.bfloat16
    n, c = pos.shape
    B = n // _P
    x = pos.reshape(B, _P, c)

    wc1, wn1 = _split_edge_weights(c1w1, c, f32)
    conv1_w = (wc1, wn1, c1b1, c1w2, c1b2, c1w3, c1b3)
    h1 = c1w3.shape[1]

    grid1 = pltpu.PrefetchScalarGridSpec(
        num_scalar_prefetch=0,
        grid=(B, _P // _Q),
        in_specs=[pl.BlockSpec((1, _P, c), lambda b, qi: (b, 0, 0))]
                 + [pl.BlockSpec(w.shape, lambda b, qi: (0, 0)) for w in conv1_w],
        out_specs=pl.BlockSpec((1, _Q, h1), lambda b, qi: (b, qi, 0)),
    )
    x1 = pl.pallas_call(
        functools.partial(_conv1_kernel, k=_K, q=_Q),
        out_shape=jax.ShapeDtypeStruct((B, _P, h1), f32),
        grid_spec=grid1,
        compiler_params=pltpu.CompilerParams(
            dimension_semantics=("parallel", "parallel"),
            vmem_limit_bytes=_VLIM),
    )(x, *conv1_w)

    wc2, wn2 = _split_edge_weights(c2w1, c2w1.shape[0] // 2, bf)
    wc2, wn2 = wc2[:h1], wn2[:h1]
    conv2_w = (wc2, wn2, c2b1, c2w2.astype(bf), c2b2, c2w3.astype(bf), c2b3)
    head_w = (w0.astype(bf), b0, l1w, l1b, l2w, l2b, l3w, l3b)
    weights = conv2_w + head_w

    grid2 = pltpu.PrefetchScalarGridSpec(
        num_scalar_prefetch=0,
        grid=(B, _P // _Q2),
        in_specs=[pl.BlockSpec((1, _P, h1), lambda b, qi: (b, 0, 0))]
                 + [pl.BlockSpec(w.shape, lambda b, qi: (0, 0)) for w in weights],
        out_specs=pl.BlockSpec((1, 1, _NC), lambda b, qi: (b, 0, 0)),
        scratch_shapes=[pltpu.VMEM((1, w0.shape[1]), f32)],
    )
    out = pl.pallas_call(
        functools.partial(_conv2_head_kernel, k=_K, q=_Q2),
        out_shape=jax.ShapeDtypeStruct((B, 1, _NC), f32),
        grid_spec=grid2,
        compiler_params=pltpu.CompilerParams(
            dimension_semantics=("parallel", "arbitrary"),
            vmem_limit_bytes=_VLIM),
    )(x1, *weights)
    return out[:, 0, :]
